# Initial kernel scaffold; baseline (speedup 1.0000x reference)
import jax
import jax.numpy as jnp
from jax.experimental import pallas as pl


def kernel(x, edge_index, edge_weight, W1_0, b1_0, W2_0, b2_0, W1_1, b1_1, W2_1, b2_1):
    raise NotImplementedError("write your pallas kernel here")


# speedup vs baseline: 2.8400x; 2.8400x over previous
import functools

import jax
import jax.numpy as jnp
from jax import lax
from jax.experimental import pallas as pl
from jax.experimental.pallas import tpu as pltpu
from jax.experimental.pallas import tpu_sc as plsc

N = 10000
D = 128
E = 320000

NC = 2
NS = 16
NW = NC * NS

CHUNK = 128
NCH = 80
G = 16
EPT = NCH * CHUNK
E_PAD = NW * EPT
N_PAD = 10240
ROWS_PT = N_PAD // NS


def _sc_segment_sum_body(h_hbm, src_hbm, dst_hbm, out_hbm,
                         sidx, didx, rows, acc, sem):
    cid = lax.axis_index("c")
    sid = lax.axis_index("s")
    wid = cid * NS + sid

    def _zrow(i, carry):
        rows[i // 8, pl.ds((i % 8) * 16, 16)] = jnp.zeros((16,), jnp.float32)
        return carry

    lax.fori_loop(0, CHUNK * 8, _zrow, 0)

    def _zcp(k, carry):
        pltpu.sync_copy(rows, acc.at[pl.ds(sid * ROWS_PT + k * CHUNK, CHUNK)])
        return carry

    lax.fori_loop(0, ROWS_PT // CHUNK, _zcp, 0)
    plsc.subcore_barrier()

    def _group(g, carry):
        pltpu.sync_copy(src_hbm.at[wid, pl.ds(g * G, G)], sidx)
        pltpu.sync_copy(dst_hbm.at[wid, pl.ds(g * G, G)], didx)

        def _chunk(j, c2):
            pltpu.async_copy(h_hbm.at[sidx.at[j]], rows, sem).wait()
            pltpu.sync_copy(rows, acc.at[didx.at[j]], add=True)
            return c2

        lax.fori_loop(0, G, _chunk, 0)
        return carry

    lax.fori_loop(0, NCH // G, _group, 0)
    plsc.subcore_barrier()

    pltpu.sync_copy(acc.at[pl.ds(sid * ROWS_PT, ROWS_PT)],
                    out_hbm.at[cid, pl.ds(sid * ROWS_PT, ROWS_PT)])


_sc_segment_sum = pl.kernel(
    _sc_segment_sum_body,
    out_type=jax.ShapeDtypeStruct((NC, N_PAD, D), jnp.float32),
    mesh=plsc.VectorSubcoreMesh(core_axis_name="c", subcore_axis_name="s"),
    scratch_types=[
        pltpu.VMEM((G, CHUNK), jnp.int32),
        pltpu.VMEM((G, CHUNK), jnp.int32),
        pltpu.VMEM((CHUNK, D), jnp.float32),
        pltpu.VMEM_SHARED((N_PAD, D), jnp.float32),
        pltpu.SemaphoreType.DMA,
    ],
)


BN = 512


def _mlp_body(last, h_ref, p0_ref, p1_ref, w1_ref, b1_ref, w2_ref, b2_ref, o_ref):
    m = h_ref[...] + p0_ref[...] + p1_ref[...]
    t = jnp.dot(m, w1_ref[...], preferred_element_type=jnp.float32,
                precision=lax.Precision.HIGHEST) + b1_ref[...]
    t = jnp.maximum(t, 0.0)
    o = jnp.dot(t, w2_ref[...], preferred_element_type=jnp.float32,
                precision=lax.Precision.HIGHEST) + b2_ref[...]
    if last:
        nrm = jnp.sqrt(jnp.sum(o * o, axis=1, keepdims=True))
        o = o / jnp.maximum(nrm, 1e-12)
    else:
        o = jnp.maximum(o, 0.0)
    o_ref[...] = o


def _mlp(h, p0, p1, w1, b1, w2, b2, last):
    row = pl.BlockSpec((BN, D), lambda i: (i, 0))
    full = pl.BlockSpec((D, D), lambda i: (0, 0))
    bias = pl.BlockSpec((1, D), lambda i: (0, 0))
    return pl.pallas_call(
        functools.partial(_mlp_body, last),
        grid=(N_PAD // BN,),
        in_specs=[row, row, row, full, bias, full, bias],
        out_specs=row,
        out_shape=jax.ShapeDtypeStruct((N_PAD, D), jnp.float32),
    )(h, p0, p1, w1, b1.reshape(1, D), w2, b2.reshape(1, D))


def kernel(x, edge_index, edge_weight, W1_0, b1_0, W2_0, b2_0,
           W1_1, b1_1, W2_1, b2_1):
    pad = E_PAD - E
    src3 = jnp.concatenate([edge_index[0], jnp.zeros((pad,), jnp.int32)]
                           ).reshape(NW, NCH, CHUNK)
    dst3 = jnp.concatenate([edge_index[1], jnp.full((pad,), N, jnp.int32)]
                           ).reshape(NW, NCH, CHUNK)
    x_pad = jnp.zeros((N_PAD, D), jnp.float32).at[:N].set(x)

    parts = _sc_segment_sum(x_pad, src3, dst3)
    h1 = _mlp(x_pad, parts[0], parts[1], W1_0, b1_0, W2_0, b2_0, last=False)
    parts = _sc_segment_sum(h1, src3, dst3)
    h2 = _mlp(h1, parts[0], parts[1], W1_1, b1_1, W2_1, b2_1, last=True)
    return h2[:N]


# speedup vs baseline: 3.1589x; 1.1123x over previous
import functools

import jax
import jax.numpy as jnp
from jax import lax
from jax.experimental import pallas as pl
from jax.experimental.pallas import tpu as pltpu
from jax.experimental.pallas import tpu_sc as plsc

N = 10000
D = 128
E = 320000

NC = 2
NS = 16
NW = NC * NS

CHUNK = 128
NCH = 80
G = 16
EPT = NCH * CHUNK
E_PAD = NW * EPT
N_PAD = 10240
ROWS_PT = N_PAD // NS


def _sc_segment_sum_body(h_hbm, src_hbm, dst_hbm, out_hbm,
                         sidx0, sidx1, didx0, didx1, r0, r1, acc,
                         semA, semB, sem_slab):
    cid = lax.axis_index("c")
    sid = lax.axis_index("s")
    wid = cid * NS + sid

    def _zrow(i, carry):
        r0[i // 8, pl.ds((i % 8) * 16, 16)] = jnp.zeros((16,), jnp.float32)
        return carry

    lax.fori_loop(0, CHUNK * 8, _zrow, 0)

    def _zcp(k, carry):
        pltpu.sync_copy(r0, acc.at[pl.ds(sid * ROWS_PT + k * CHUNK, CHUNK)])
        return carry

    lax.fori_loop(0, ROWS_PT // CHUNK, _zcp, 0)
    plsc.subcore_barrier()

    NGRP = NCH // G
    slabs = [(sidx0, didx0), (sidx1, didx1)]
    pend = None
    for g in range(NGRP):
        sbuf, dbuf = slabs[g % 2]
        if g == 0:
            pltpu.sync_copy(src_hbm.at[wid, pl.ds(0, G)], sbuf)
            pltpu.sync_copy(dst_hbm.at[wid, pl.ds(0, G)], dbuf)
        else:
            for c in pend:
                c.wait()
        if g + 1 < NGRP:
            nsb, ndb = slabs[(g + 1) % 2]
            pend = (
                pltpu.async_copy(src_hbm.at[wid, pl.ds((g + 1) * G, G)],
                                 nsb, sem_slab),
                pltpu.async_copy(dst_hbm.at[wid, pl.ds((g + 1) * G, G)],
                                 ndb, sem_slab),
            )

        pltpu.async_copy(h_hbm.at[sbuf.at[0]], r0, semA)
        pltpu.async_copy(h_hbm.at[sbuf.at[1]], r1, semB)

        def _pair(s, carry, sbuf=sbuf, dbuf=dbuf):
            pltpu.make_async_copy(h_hbm.at[sbuf.at[2 * s]], r0, semA).wait()
            pltpu.sync_copy(r0, acc.at[dbuf.at[2 * s]], add=True)

            @pl.when(s < G // 2 - 1)
            def _():
                pltpu.async_copy(h_hbm.at[sbuf.at[2 * s + 2]], r0, semA)

            pltpu.make_async_copy(h_hbm.at[sbuf.at[2 * s + 1]], r1, semB).wait()
            pltpu.sync_copy(r1, acc.at[dbuf.at[2 * s + 1]], add=True)

            @pl.when(s < G // 2 - 1)
            def _():
                pltpu.async_copy(h_hbm.at[sbuf.at[2 * s + 3]], r1, semB)

            return carry

        lax.fori_loop(0, G // 2, _pair, 0)

    plsc.subcore_barrier()

    pltpu.sync_copy(acc.at[pl.ds(sid * ROWS_PT, ROWS_PT)],
                    out_hbm.at[cid, pl.ds(sid * ROWS_PT, ROWS_PT)])


_sc_segment_sum = pl.kernel(
    _sc_segment_sum_body,
    out_type=jax.ShapeDtypeStruct((NC, N_PAD, D), jnp.float32),
    mesh=plsc.VectorSubcoreMesh(core_axis_name="c", subcore_axis_name="s"),
    scratch_types=[
        pltpu.VMEM((G, CHUNK), jnp.int32),
        pltpu.VMEM((G, CHUNK), jnp.int32),
        pltpu.VMEM((G, CHUNK), jnp.int32),
        pltpu.VMEM((G, CHUNK), jnp.int32),
        pltpu.VMEM((CHUNK, D), jnp.float32),
        pltpu.VMEM((CHUNK, D), jnp.float32),
        pltpu.VMEM_SHARED((N_PAD, D), jnp.float32),
        pltpu.SemaphoreType.DMA,
        pltpu.SemaphoreType.DMA,
        pltpu.SemaphoreType.DMA,
    ],
)


BN = 512


def _mlp_body(last, h_ref, p0_ref, p1_ref, w1_ref, b1_ref, w2_ref, b2_ref, o_ref):
    m = h_ref[...] + p0_ref[...] + p1_ref[...]
    t = jnp.dot(m, w1_ref[...], preferred_element_type=jnp.float32,
                precision=lax.Precision.HIGHEST) + b1_ref[...]
    t = jnp.maximum(t, 0.0)
    o = jnp.dot(t, w2_ref[...], preferred_element_type=jnp.float32,
                precision=lax.Precision.HIGHEST) + b2_ref[...]
    if last:
        nrm = jnp.sqrt(jnp.sum(o * o, axis=1, keepdims=True))
        o = o / jnp.maximum(nrm, 1e-12)
    else:
        o = jnp.maximum(o, 0.0)
    o_ref[...] = o


def _mlp(h, p0, p1, w1, b1, w2, b2, last):
    row = pl.BlockSpec((BN, D), lambda i: (i, 0))
    full = pl.BlockSpec((D, D), lambda i: (0, 0))
    bias = pl.BlockSpec((1, D), lambda i: (0, 0))
    return pl.pallas_call(
        functools.partial(_mlp_body, last),
        grid=(N_PAD // BN,),
        in_specs=[row, row, row, full, bias, full, bias],
        out_specs=row,
        out_shape=jax.ShapeDtypeStruct((N_PAD, D), jnp.float32),
    )(h, p0, p1, w1, b1.reshape(1, D), w2, b2.reshape(1, D))


def kernel(x, edge_index, edge_weight, W1_0, b1_0, W2_0, b2_0,
           W1_1, b1_1, W2_1, b2_1):
    pad = E_PAD - E
    src3 = jnp.concatenate([edge_index[0], jnp.zeros((pad,), jnp.int32)]
                           ).reshape(NW, NCH, CHUNK)
    dst3 = jnp.concatenate([edge_index[1], jnp.full((pad,), N, jnp.int32)]
                           ).reshape(NW, NCH, CHUNK)
    x_pad = jnp.zeros((N_PAD, D), jnp.float32).at[:N].set(x)

    parts = _sc_segment_sum(x_pad, src3, dst3)
    h1 = _mlp(x_pad, parts[0], parts[1], W1_0, b1_0, W2_0, b2_0, last=False)
    parts = _sc_segment_sum(h1, src3, dst3)
    h2 = _mlp(h1, parts[0], parts[1], W1_1, b1_1, W2_1, b2_1, last=True)
    return h2[:N]
